# baseline (speedup 1.0000x reference)
import functools

import jax
import jax.numpy as jnp
from jax import lax
from jax.experimental import pallas as pl
from jax.experimental.pallas import tpu as pltpu
from jax.experimental.pallas import tpu_sc as plsc

N, D, A = 1024, 64, 256
NW = 32
TB = N // NW
L = 16
NJ = D // L
SPLIT = 128
UCH = min(256, SPLIT)
TP = 4
TTB = 512
SSB = 128


def _prep_body(h_ref, w1_ref, w2_ref, b_ref, e_ref, a_ref,
               u_ref, v_ref, vt_ref, et_ref, at_ref):
    h = h_ref[...]
    dn = (((1,), (1,)), ((), ()))
    u_ref[...] = lax.dot_general(h, w1_ref[...], dn,
                                 preferred_element_type=jnp.float32) + b_ref[...]
    v = lax.dot_general(h, w2_ref[...], dn, preferred_element_type=jnp.float32)
    v_ref[...] = v
    vt_ref[...] = v.T
    et_ref[...] = e_ref[0:SPLIT, :].T
    at_ref[...] = a_ref[0:SPLIT, :].T


_prep = pl.pallas_call(
    _prep_body,
    out_shape=[jax.ShapeDtypeStruct((N, D), jnp.float32),
               jax.ShapeDtypeStruct((N, D), jnp.float32),
               jax.ShapeDtypeStruct((D, N), jnp.float32),
               jax.ShapeDtypeStruct((N, SPLIT), jnp.float32),
               jax.ShapeDtypeStruct((N, SPLIT), jnp.float32)],
)


def _agg_body(u_hbm, v_hbm, w3_hbm, et_hbm, at_hbm, out_hbm,
              v_v, u_v, et_v, at_v, acc_v, w3_v):
    wid = lax.axis_index("s") * 2 + lax.axis_index("c")
    t0 = wid * TB

    pltpu.sync_copy(v_hbm.at[pl.ds(t0, TB)], v_v)
    pltpu.sync_copy(w3_hbm, w3_v)
    pltpu.sync_copy(et_hbm.at[pl.ds(t0, TB)], et_v)
    pltpu.sync_copy(at_hbm.at[pl.ds(t0, TB)], at_v)

    w3 = tuple(w3_v[0, pl.ds(j * L, L)] for j in range(NJ))

    for half in range(SPLIT // UCH):
        pltpu.sync_copy(u_hbm.at[pl.ds(half * UCH, UCH)], u_v)

        def t_body(tb, _, half=half):
            t0l = tb * TP
            vt = [[v_v[t0l + p, pl.ds(j * L, L)] for j in range(NJ)]
                  for p in range(TP)]
            if half == 0:
                acc0 = tuple(jnp.zeros((L,), jnp.float32)
                             for _ in range(TP * NJ))
            else:
                acc0 = tuple(acc_v[t0l + p, pl.ds(j * L, L)]
                             for p in range(TP) for j in range(NJ))

            def s_body(sb, acc, half=half):
                s0 = half * UCH + sb * L
                e16 = [et_v[t0l + p, pl.ds(s0, L)] for p in range(TP)]
                a16 = [at_v[t0l + p, pl.ds(s0, L)] for p in range(TP)]
                acc = [list(acc[p * NJ:(p + 1) * NJ]) for p in range(TP)]
                for i in range(L):
                    u = [u_v[sb * L + i, pl.ds(j * L, L)] for j in range(NJ)]
                    for p in range(TP):
                        e = e16[p][i]
                        a = a16[p][i]
                        for j in range(NJ):
                            m = jnp.maximum(u[j] + (vt[p][j] + e * w3[j]), 0.0)
                            acc[p][j] = acc[p][j] + m * a
                return tuple(acc[p][j] for p in range(TP) for j in range(NJ))

            acc = lax.fori_loop(0, UCH // L, s_body, acc0)
            for p in range(TP):
                for j in range(NJ):
                    acc_v[t0l + p, pl.ds(j * L, L)] = acc[p * NJ + j]
            return 0

        lax.fori_loop(0, TB // TP, t_body, 0)

    pltpu.sync_copy(acc_v, out_hbm.at[pl.ds(t0, TB)])


@functools.cache
def _make_agg():
    return functools.partial(
        pl.kernel,
        out_type=jax.ShapeDtypeStruct((N, D), jnp.float32),
        mesh=plsc.VectorSubcoreMesh(core_axis_name="c", subcore_axis_name="s"),
        scratch_types=[
            pltpu.VMEM((TB, D), jnp.float32),
            pltpu.VMEM((UCH, D), jnp.float32),
            pltpu.VMEM((TB, SPLIT), jnp.float32),
            pltpu.VMEM((TB, SPLIT), jnp.float32),
            pltpu.VMEM((TB, D), jnp.float32),
            pltpu.VMEM((1, D), jnp.float32),
        ],
    )(_agg_body)


def _tcagg_body(u_ref, e_ref, a_ref, vt_ref, w3_ref, out_ref):
    si = pl.program_id(1)

    @pl.when(si == 0)
    def _():
        out_ref[...] = jnp.zeros_like(out_ref)

    w3v = w3_ref[...]
    rows = []
    for d in range(D):
        vtb = vt_ref[d:d + 1, :]
        w3d = w3v[0, d]
        acc = jnp.zeros((8, TTB), jnp.float32)
        for ss in range(0, SSB, 8):
            pre = u_ref[ss:ss + 8, d:d + 1] + vtb + e_ref[ss:ss + 8, :] * w3d
            acc = acc + jnp.maximum(pre, 0.0) * a_ref[ss:ss + 8, :]
        rows.append(jnp.sum(acc, axis=0, keepdims=True))
    out_ref[...] += jnp.concatenate(rows, axis=0)


_tcagg = pl.pallas_call(
    _tcagg_body,
    grid=(N // TTB, (N - SPLIT) // SSB),
    in_specs=[
        pl.BlockSpec((SSB, D), lambda ti, si: (si + SPLIT // SSB, 0)),
        pl.BlockSpec((SSB, TTB), lambda ti, si: (si + SPLIT // SSB, ti)),
        pl.BlockSpec((SSB, TTB), lambda ti, si: (si + SPLIT // SSB, ti)),
        pl.BlockSpec((D, TTB), lambda ti, si: (0, ti)),
        pl.BlockSpec((1, D), lambda ti, si: (0, 0)),
    ],
    out_specs=pl.BlockSpec((D, TTB), lambda ti, si: (0, ti)),
    out_shape=jax.ShapeDtypeStruct((D, N), jnp.float32),
)


def _head_body(aggs_ref, aggt_ref, h_ref, wih_ref, whh_ref, bih_ref, bhh_ref,
               wpol_ref, bpol_ref, out_ref):
    agg = aggs_ref[...] + aggt_ref[...].T
    h = h_ref[...]
    dn = (((1,), (1,)), ((), ()))
    gi = lax.dot_general(agg, wih_ref[...], dn,
                         preferred_element_type=jnp.float32) + bih_ref[...]
    gh = lax.dot_general(h, whh_ref[...], dn,
                         preferred_element_type=jnp.float32) + bhh_ref[...]
    r = jax.nn.sigmoid(gi[:, :D] + gh[:, :D])
    z = jax.nn.sigmoid(gi[:, D:2 * D] + gh[:, D:2 * D])
    n = jnp.tanh(gi[:, 2 * D:] + r * gh[:, 2 * D:])
    upd = (1.0 - z) * n + z * h
    s = jnp.sum(upd, axis=0, keepdims=True)
    out_ref[...] = lax.dot_general(s, wpol_ref[...], dn,
                                   preferred_element_type=jnp.float32) + bpol_ref[...]


_head = pl.pallas_call(
    _head_body,
    out_shape=jax.ShapeDtypeStruct((1, A), jnp.float32),
)


def kernel(node_features, edge_features, adjacency_matrix, W_msg, b_msg,
           W_ih, W_hh, b_ih, b_hh, W_pol, b_pol):
    w1 = W_msg[:, :D]
    w2 = W_msg[:, D:2 * D]
    w3 = W_msg[:, 2 * D].reshape(1, D)
    adjf = adjacency_matrix.astype(jnp.float32)
    u, v, vt, et, at = _prep(node_features, w1, w2, b_msg.reshape(1, D),
                             edge_features, adjf)
    agg_tc = _tcagg(u, edge_features, adjf, vt, w3)
    agg_sc = _make_agg()(u, v, w3, et, at)
    out = _head(agg_sc, agg_tc, node_features, W_ih, W_hh,
                b_ih.reshape(1, 3 * D), b_hh.reshape(1, 3 * D), W_pol,
                b_pol.reshape(1, A))
    return out.reshape(A)


# speedup vs baseline: 1.1392x; 1.1392x over previous
import functools

import jax
import jax.numpy as jnp
from jax import lax
from jax.experimental import pallas as pl
from jax.experimental.pallas import tpu as pltpu
from jax.experimental.pallas import tpu_sc as plsc

N, D, A = 1024, 64, 256
NW = 32
TB = N // NW
L = 16
NJ = D // L
SPLIT = 256
UCH = min(256, SPLIT)
TP = 4
TTB = 512
SSB = 256


def _prep_body(h_ref, w1_ref, w2_ref, b_ref, e_ref, a_ref,
               u_ref, v_ref, vt_ref, et_ref, at_ref):
    h = h_ref[...]
    dn = (((1,), (1,)), ((), ()))
    u_ref[...] = lax.dot_general(h, w1_ref[...], dn,
                                 preferred_element_type=jnp.float32) + b_ref[...]
    v = lax.dot_general(h, w2_ref[...], dn, preferred_element_type=jnp.float32)
    v_ref[...] = v
    vt_ref[...] = v.T
    et_ref[...] = e_ref[0:SPLIT, :].T
    at_ref[...] = a_ref[0:SPLIT, :].T


_prep = pl.pallas_call(
    _prep_body,
    out_shape=[jax.ShapeDtypeStruct((N, D), jnp.float32),
               jax.ShapeDtypeStruct((N, D), jnp.float32),
               jax.ShapeDtypeStruct((D, N), jnp.float32),
               jax.ShapeDtypeStruct((N, SPLIT), jnp.float32),
               jax.ShapeDtypeStruct((N, SPLIT), jnp.float32)],
)


def _agg_body(u_hbm, v_hbm, w3_hbm, et_hbm, at_hbm, out_hbm,
              v_v, u_v, et_v, at_v, acc_v, w3_v):
    wid = lax.axis_index("s") * 2 + lax.axis_index("c")
    t0 = wid * TB

    pltpu.sync_copy(v_hbm.at[pl.ds(t0, TB)], v_v)
    pltpu.sync_copy(w3_hbm, w3_v)
    pltpu.sync_copy(et_hbm.at[pl.ds(t0, TB)], et_v)
    pltpu.sync_copy(at_hbm.at[pl.ds(t0, TB)], at_v)

    w3 = tuple(w3_v[0, pl.ds(j * L, L)] for j in range(NJ))

    for half in range(SPLIT // UCH):
        pltpu.sync_copy(u_hbm.at[pl.ds(half * UCH, UCH)], u_v)

        def t_body(tb, _, half=half):
            t0l = tb * TP
            vt = [[v_v[t0l + p, pl.ds(j * L, L)] for j in range(NJ)]
                  for p in range(TP)]
            if half == 0:
                acc0 = tuple(jnp.zeros((L,), jnp.float32)
                             for _ in range(TP * NJ))
            else:
                acc0 = tuple(acc_v[t0l + p, pl.ds(j * L, L)]
                             for p in range(TP) for j in range(NJ))

            def s_body(sb, acc, half=half):
                s0 = half * UCH + sb * L
                e16 = [et_v[t0l + p, pl.ds(s0, L)] for p in range(TP)]
                a16 = [at_v[t0l + p, pl.ds(s0, L)] for p in range(TP)]
                acc = [list(acc[p * NJ:(p + 1) * NJ]) for p in range(TP)]
                for i in range(L):
                    u = [u_v[sb * L + i, pl.ds(j * L, L)] for j in range(NJ)]
                    for p in range(TP):
                        e = e16[p][i]
                        a = a16[p][i]
                        for j in range(NJ):
                            m = jnp.maximum(u[j] + (vt[p][j] + e * w3[j]), 0.0)
                            acc[p][j] = acc[p][j] + m * a
                return tuple(acc[p][j] for p in range(TP) for j in range(NJ))

            acc = lax.fori_loop(0, UCH // L, s_body, acc0)
            for p in range(TP):
                for j in range(NJ):
                    acc_v[t0l + p, pl.ds(j * L, L)] = acc[p * NJ + j]
            return 0

        lax.fori_loop(0, TB // TP, t_body, 0)

    pltpu.sync_copy(acc_v, out_hbm.at[pl.ds(t0, TB)])


@functools.cache
def _make_agg():
    return functools.partial(
        pl.kernel,
        out_type=jax.ShapeDtypeStruct((N, D), jnp.float32),
        mesh=plsc.VectorSubcoreMesh(core_axis_name="c", subcore_axis_name="s"),
        scratch_types=[
            pltpu.VMEM((TB, D), jnp.float32),
            pltpu.VMEM((UCH, D), jnp.float32),
            pltpu.VMEM((TB, SPLIT), jnp.float32),
            pltpu.VMEM((TB, SPLIT), jnp.float32),
            pltpu.VMEM((TB, D), jnp.float32),
            pltpu.VMEM((1, D), jnp.float32),
        ],
    )(_agg_body)


def _tcagg_body(u_ref, e_ref, a_ref, vt_ref, w3_ref, out_ref):
    si = pl.program_id(1)

    @pl.when(si == 0)
    def _():
        out_ref[...] = jnp.zeros_like(out_ref)

    w3v = w3_ref[...]
    rows = []
    for d in range(D):
        vtb = vt_ref[d:d + 1, :]
        w3d = w3v[0, d]
        acc = jnp.zeros((8, TTB), jnp.float32)
        for ss in range(0, SSB, 8):
            pre = u_ref[ss:ss + 8, d:d + 1] + vtb + e_ref[ss:ss + 8, :] * w3d
            acc = acc + jnp.maximum(pre, 0.0) * a_ref[ss:ss + 8, :]
        rows.append(jnp.sum(acc, axis=0, keepdims=True))
    out_ref[...] += jnp.concatenate(rows, axis=0)


_tcagg = pl.pallas_call(
    _tcagg_body,
    grid=(N // TTB, (N - SPLIT) // SSB),
    in_specs=[
        pl.BlockSpec((SSB, D), lambda ti, si: (si + SPLIT // SSB, 0)),
        pl.BlockSpec((SSB, TTB), lambda ti, si: (si + SPLIT // SSB, ti)),
        pl.BlockSpec((SSB, TTB), lambda ti, si: (si + SPLIT // SSB, ti)),
        pl.BlockSpec((D, TTB), lambda ti, si: (0, ti)),
        pl.BlockSpec((1, D), lambda ti, si: (0, 0)),
    ],
    out_specs=pl.BlockSpec((D, TTB), lambda ti, si: (0, ti)),
    out_shape=jax.ShapeDtypeStruct((D, N), jnp.float32),
)


def _head_body(aggs_ref, aggt_ref, h_ref, wih_ref, whh_ref, bih_ref, bhh_ref,
               wpol_ref, bpol_ref, out_ref):
    agg = aggs_ref[...] + aggt_ref[...].T
    h = h_ref[...]
    dn = (((1,), (1,)), ((), ()))
    gi = lax.dot_general(agg, wih_ref[...], dn,
                         preferred_element_type=jnp.float32) + bih_ref[...]
    gh = lax.dot_general(h, whh_ref[...], dn,
                         preferred_element_type=jnp.float32) + bhh_ref[...]
    r = jax.nn.sigmoid(gi[:, :D] + gh[:, :D])
    z = jax.nn.sigmoid(gi[:, D:2 * D] + gh[:, D:2 * D])
    n = jnp.tanh(gi[:, 2 * D:] + r * gh[:, 2 * D:])
    upd = (1.0 - z) * n + z * h
    s = jnp.sum(upd, axis=0, keepdims=True)
    out_ref[...] = lax.dot_general(s, wpol_ref[...], dn,
                                   preferred_element_type=jnp.float32) + bpol_ref[...]


_head = pl.pallas_call(
    _head_body,
    out_shape=jax.ShapeDtypeStruct((1, A), jnp.float32),
)


def kernel(node_features, edge_features, adjacency_matrix, W_msg, b_msg,
           W_ih, W_hh, b_ih, b_hh, W_pol, b_pol):
    w1 = W_msg[:, :D]
    w2 = W_msg[:, D:2 * D]
    w3 = W_msg[:, 2 * D].reshape(1, D)
    adjf = adjacency_matrix.astype(jnp.float32)
    u, v, vt, et, at = _prep(node_features, w1, w2, b_msg.reshape(1, D),
                             edge_features, adjf)
    agg_tc = _tcagg(u, edge_features, adjf, vt, w3)
    agg_sc = _make_agg()(u, v, w3, et, at)
    out = _head(agg_sc, agg_tc, node_features, W_ih, W_hh,
                b_ih.reshape(1, 3 * D), b_hh.reshape(1, 3 * D), W_pol,
                b_pol.reshape(1, A))
    return out.reshape(A)
